# baseline (speedup 1.0000x reference)
import functools

import jax
import jax.numpy as jnp
from jax import lax
from jax.experimental import pallas as pl
from jax.experimental.pallas import tpu as pltpu
from jax.experimental.pallas import tpu_sc as plsc

B = 8
W = 32
H = 32
DIM = 384
LANES = 16
NCHUNK = DIM // LANES
NC = 2
NS = 16
HH = H // 2


def _body(emb_x_hbm, emb_y_hbm, out_hbm, ex_v, ey_v, z_v, insem, outsem):
    wid = lax.axis_index("s") * NC + lax.axis_index("c")
    ld_ex = pltpu.async_copy(emb_x_hbm.at[wid], ex_v, insem)
    ld_ey = pltpu.async_copy(emb_y_hbm.at[pl.ds(0, H)], ey_v, insem)
    ld_ex.wait()
    ld_ey.wait()

    def compute_rows(y0):
        for c in range(NCHUNK):
            sl = pl.ds(c * LANES, LANES)
            vx = ex_v[sl]
            for y in range(y0, y0 + HH):
                z_v[y, sl] = vx * ey_v[y, sl]

    compute_rows(0)
    first = [
        pltpu.async_copy(
            z_v.at[pl.ds(0, HH)], out_hbm.at[b, pl.ds(wid * H, HH)], outsem
        )
        for b in range(B)
    ]
    compute_rows(HH)
    second = [
        pltpu.async_copy(
            z_v.at[pl.ds(HH, HH)], out_hbm.at[b, pl.ds(wid * H + HH, HH)], outsem
        )
        for b in range(B)
    ]
    for cp in first + second:
        cp.wait()


@jax.jit
def _position_embedding(emb_x_table, emb_y_table):
    mesh = plsc.VectorSubcoreMesh(
        core_axis_name="c", subcore_axis_name="s", num_cores=NC, num_subcores=NS
    )
    run = functools.partial(
        pl.kernel,
        out_type=jax.ShapeDtypeStruct((B, W * H, DIM), jnp.float32),
        mesh=mesh,
        scratch_types=[
            pltpu.VMEM((DIM,), jnp.float32),
            pltpu.VMEM((H, DIM), jnp.float32),
            pltpu.VMEM((H, DIM), jnp.float32),
            pltpu.SemaphoreType.DMA,
            pltpu.SemaphoreType.DMA,
        ],
    )(_body)
    return run(emb_x_table, emb_y_table)


def kernel(patches, emb_x_table, emb_y_table):
    del patches
    return _position_embedding(emb_x_table, emb_y_table)


# speedup vs baseline: 1.0458x; 1.0458x over previous
import functools

import jax
import jax.numpy as jnp
from jax import lax
from jax.experimental import pallas as pl
from jax.experimental.pallas import tpu as pltpu
from jax.experimental.pallas import tpu_sc as plsc

B = 8
W = 32
H = 32
DIM = 384
LANES = 16
NCHUNK = DIM // LANES
NC = 2
NS = 16


def _body(emb_x_hbm, emb_y_hbm, out_hbm, ex_v, ey_v, z_v, insem, outsem):
    wid = lax.axis_index("s") * NC + lax.axis_index("c")
    ld_ex = pltpu.async_copy(emb_x_hbm.at[wid], ex_v, insem)
    ld_ey = pltpu.async_copy(emb_y_hbm.at[pl.ds(0, H)], ey_v, insem)
    ld_ex.wait()
    ld_ey.wait()

    def yloop(y, carry):
        for c in range(NCHUNK):
            sl = pl.ds(c * LANES, LANES)
            z_v[y, sl] = ex_v[sl] * ey_v[y, sl]
        return carry

    lax.fori_loop(0, H, yloop, 0)

    copies = [
        pltpu.async_copy(z_v, out_hbm.at[b, pl.ds(wid * H, H)], outsem)
        for b in range(B)
    ]
    for cp in copies:
        cp.wait()


@jax.jit
def _position_embedding(emb_x_table, emb_y_table):
    mesh = plsc.VectorSubcoreMesh(
        core_axis_name="c", subcore_axis_name="s", num_cores=NC, num_subcores=NS
    )
    run = functools.partial(
        pl.kernel,
        out_type=jax.ShapeDtypeStruct((B, W * H, DIM), jnp.float32),
        mesh=mesh,
        scratch_types=[
            pltpu.VMEM((DIM,), jnp.float32),
            pltpu.VMEM((H, DIM), jnp.float32),
            pltpu.VMEM((H, DIM), jnp.float32),
            pltpu.SemaphoreType.DMA,
            pltpu.SemaphoreType.DMA,
        ],
    )(_body)
    return run(emb_x_table, emb_y_table)


def kernel(patches, emb_x_table, emb_y_table):
    del patches
    return _position_embedding(emb_x_table, emb_y_table)


# speedup vs baseline: 1.0920x; 1.0442x over previous
import functools

import jax
import jax.numpy as jnp
from jax import lax
from jax.experimental import pallas as pl
from jax.experimental.pallas import tpu as pltpu
from jax.experimental.pallas import tpu_sc as plsc

B = 8
W = 32
H = 32
DIM = 384
LANES = 16
NCHUNK = DIM // LANES
NC = 2
NS = 16


HH = H // 2


def _body(emb_x_hbm, emb_y_hbm, out_hbm, ex_v, ey_v, z_v, insem, insem2, outsem):
    wid = lax.axis_index("s") * NC + lax.axis_index("c")
    ld_ex = pltpu.async_copy(emb_x_hbm.at[wid], ex_v, insem)
    ld_ey0 = pltpu.async_copy(
        emb_y_hbm.at[pl.ds(0, HH)], ey_v.at[pl.ds(0, HH)], insem
    )
    ld_ey1 = pltpu.async_copy(
        emb_y_hbm.at[pl.ds(HH, HH)], ey_v.at[pl.ds(HH, HH)], insem2
    )
    ld_ex.wait()
    ld_ey0.wait()

    def yloop(y, carry):
        for c in range(NCHUNK):
            sl = pl.ds(c * LANES, LANES)
            z_v[y, sl] = ex_v[sl] * ey_v[y, sl]
        return carry

    lax.fori_loop(0, HH, yloop, 0)
    first = [
        pltpu.async_copy(
            z_v.at[pl.ds(0, HH)], out_hbm.at[b, pl.ds(wid * H, HH)], outsem
        )
        for b in range(B)
    ]
    ld_ey1.wait()
    lax.fori_loop(HH, H, yloop, 0)
    second = [
        pltpu.async_copy(
            z_v.at[pl.ds(HH, HH)], out_hbm.at[b, pl.ds(wid * H + HH, HH)], outsem
        )
        for b in range(B)
    ]
    for cp in first + second:
        cp.wait()


@jax.jit
def _position_embedding(emb_x_table, emb_y_table):
    mesh = plsc.VectorSubcoreMesh(
        core_axis_name="c", subcore_axis_name="s", num_cores=NC, num_subcores=NS
    )
    run = functools.partial(
        pl.kernel,
        out_type=jax.ShapeDtypeStruct((B, W * H, DIM), jnp.float32),
        mesh=mesh,
        scratch_types=[
            pltpu.VMEM((DIM,), jnp.float32),
            pltpu.VMEM((H, DIM), jnp.float32),
            pltpu.VMEM((H, DIM), jnp.float32),
            pltpu.SemaphoreType.DMA,
            pltpu.SemaphoreType.DMA,
            pltpu.SemaphoreType.DMA,
        ],
    )(_body)
    return run(emb_x_table, emb_y_table)


def kernel(patches, emb_x_table, emb_y_table):
    del patches
    return _position_embedding(emb_x_table, emb_y_table)
